# baseline (speedup 1.0000x reference)
import functools

import jax
import jax.numpy as jnp
from jax import lax
from jax.experimental import pallas as pl
from jax.experimental.pallas import tpu as pltpu
from jax.experimental.pallas import tpu_sc as plsc

NUM_EMBEDDINGS = 100000
EMBEDDING_DIM = 64
BATCH = 1024

_NC = 2
_NS = 16
_NW = _NC * _NS
_NGROUP = 8
_GB = BATCH // _NGROUP
_NQ = 4
_QF = EMBEDDING_DIM // _NQ


@functools.partial(
    pl.kernel,
    mesh=plsc.VectorSubcoreMesh(core_axis_name="c", subcore_axis_name="s"),
    out_type=jax.ShapeDtypeStruct((EMBEDDING_DIM, BATCH), jnp.float32),
    scratch_types=[
        pltpu.VMEM((_GB,), jnp.int32),
        pltpu.VMEM((_GB,), jnp.int32),
        pltpu.VMEM((_GB, 128), jnp.float32),
        pltpu.VMEM((_QF, _GB), jnp.float32),
        pltpu.SemaphoreType.DMA,
    ],
    compiler_params=pltpu.CompilerParams(needs_layout_passes=False),
)
def _gather_packed(idx_hbm, w2_hbm, outt_hbm, idx_v, pidx_v, rows_v, out_v, sem):
    wid = lax.axis_index("s") * _NC + lax.axis_index("c")
    g = wid // _NQ
    q = wid % _NQ
    base = g * _GB
    pltpu.sync_copy(idx_hbm.at[pl.ds(base, _GB)], idx_v)
    for k in range(_GB // 16):
        xi = idx_v[pl.ds(k * 16, 16)]
        pidx_v[pl.ds(k * 16, 16)] = xi >> 1
    pltpu.async_copy(w2_hbm.at[pidx_v], rows_v, sem).wait()
    lane = lax.iota(jnp.int32, 16)
    for k in range(_GB // 16):
        xi = idx_v[pl.ds(k * 16, 16)]
        col0 = (xi & 1) * 64 + q * _QF
        row_ids = lane + (k * 16)
        for f in range(_QF):
            vals = plsc.load_gather(rows_v, [row_ids, col0 + f])
            out_v[f, pl.ds(k * 16, 16)] = vals
    pltpu.sync_copy(out_v, outt_hbm.at[pl.ds(q * _QF, _QF), pl.ds(base, _GB)])


def kernel(x, W):
    outt = _gather_packed(x.astype(jnp.int32), W.reshape(NUM_EMBEDDINGS // 2, 128))
    return outt.T
